# baseline (speedup 1.0000x reference)
import functools

import jax
import jax.numpy as jnp
from jax import lax
from jax.experimental import pallas as pl
from jax.experimental.pallas import tpu as pltpu
from jax.experimental.pallas import tpu_sc as plsc

N = 10000
E = 320000
D = 128
HD = D // 2

NC = 2
NS = 16
NW = NC * NS
R = 10240
STRIPE = R // NS

EPT = 20480
CH = EPT // 128
CHD = (EPT * NS) // (NW * 128)

_mesh = plsc.VectorSubcoreMesh(core_axis_name="c", subcore_axis_name="s")


@functools.partial(
    pl.kernel,
    out_type=jax.ShapeDtypeStruct((NC, R), jnp.float32),
    mesh=_mesh,
    scratch_types=[
        pltpu.VMEM((CHD, 128), jnp.int32),
        pltpu.VMEM((STRIPE,), jnp.float32),
        pltpu.VMEM((128,), jnp.float32),
        pltpu.VMEM_SHARED((R,), jnp.float32),
    ],
)
def _sc_deg(dstb_hbm, out_hbm, dst_v, stage_v, ones_v, deg_sh):
    cid = lax.axis_index("c")
    sid = lax.axis_index("s")
    wid = cid * NS + sid
    pltpu.sync_copy(dstb_hbm.at[wid], dst_v)

    def _zero(i, _):
        stage_v[pl.ds(i * 16, 16)] = jnp.zeros((16,), jnp.float32)
        return 0

    lax.fori_loop(0, STRIPE // 16, _zero, 0)
    for k in range(8):
        ones_v[pl.ds(k * 16, 16)] = jnp.ones((16,), jnp.float32)
    pltpu.sync_copy(stage_v, deg_sh.at[pl.ds(sid * STRIPE, STRIPE)])
    plsc.subcore_barrier()

    def _body(j, _):
        pltpu.sync_copy(ones_v, deg_sh.at[dst_v.at[j]], add=True)
        return 0

    lax.fori_loop(0, CHD, _body, 0)
    plsc.subcore_barrier()
    pltpu.sync_copy(deg_sh.at[pl.ds(sid * STRIPE, STRIPE)], stage_v)
    pltpu.sync_copy(stage_v, out_hbm.at[cid, pl.ds(sid * STRIPE, STRIPE)])


@functools.partial(
    pl.kernel,
    out_type=jax.ShapeDtypeStruct((NC, R, HD), jnp.float32),
    mesh=_mesh,
    scratch_types=[
        pltpu.VMEM((CH, 128), jnp.int32),
        pltpu.VMEM((CH, 128), jnp.int32),
        pltpu.VMEM((128, HD), jnp.float32),
        pltpu.VMEM((128, HD), jnp.float32),
        pltpu.VMEM((128, HD), jnp.float32),
        pltpu.VMEM((128, HD), jnp.float32),
        pltpu.VMEM_SHARED((R, HD), jnp.float32),
        [pltpu.SemaphoreType.DMA] * 4,
        [pltpu.SemaphoreType.DMA] * 4,
        [pltpu.SemaphoreType.DMA] * 4,
    ],
    compiler_params=pltpu.CompilerParams(use_tc_tiling_on_sc=False),
)
def _sc_prop(hs_hbm, srcb_hbm, dstb_hbm, out_hbm, src_v, dst_v, rows0, rows1,
             rows2, rows3, acc_sh, sg, ss, rs):
    cid = lax.axis_index("c")
    sid = lax.axis_index("s")
    pltpu.sync_copy(srcb_hbm.at[cid, sid], src_v)
    pltpu.sync_copy(dstb_hbm.at[sid], dst_v)

    def _zero(i, _):
        for k in range(HD // 16):
            rows0[i, pl.ds(k * 16, 16)] = jnp.zeros((16,), jnp.float32)
        return 0

    lax.fori_loop(0, 128, _zero, 0)
    base = sid * STRIPE
    for t in range(STRIPE // 128):
        pltpu.sync_copy(rows0, acc_sh.at[pl.ds(base + t * 128, 128), :])
    plsc.subcore_barrier()

    rows = (rows0, rows1, rows2, rows3)

    def _gather(t, b):
        return pltpu.async_copy(hs_hbm.at[src_v.at[t]], rows[b], sg[b])

    def _wait_gather(t, b):
        pltpu.make_async_copy(hs_hbm.at[src_v.at[t]], rows[b], sg[b]).wait()

    def _scatter(t, b):
        return pltpu.async_copy(rows[b], acc_sh.at[dst_v.at[t]], ss[b],
                                add=True)

    def _wait_scatter(t, b):
        pltpu.make_async_copy(rows[b], acc_sh.at[dst_v.at[t]], ss[b]).wait()

    _gather(0, 0)
    _gather(1, 1)
    _gather(2, 2)
    _wait_gather(0, 0)
    _scatter(0, 0)
    _gather(3, 3)
    _wait_gather(1, 1)
    _scatter(1, 1)

    def _body(i, _):
        g = 4 * i
        for b in range(4):
            t = g + b
            b2 = (b + 2) % 4
            _wait_scatter(t - 4, b)
            _gather(t, b)
            _wait_gather(t - 2, b2)
            _scatter(t - 2, b2)
        return 0

    lax.fori_loop(1, CH // 4, _body, 0)
    _wait_gather(CH - 2, (CH - 2) % 4)
    _scatter(CH - 2, (CH - 2) % 4)
    _wait_gather(CH - 1, (CH - 1) % 4)
    _scatter(CH - 1, (CH - 1) % 4)
    for b in range(4):
        _wait_scatter(CH - 4 + b, b)
    plsc.subcore_barrier()

    nrd = STRIPE // 128
    rd = [None] * nrd
    wr = [None] * nrd
    rd[0] = pltpu.async_copy(acc_sh.at[pl.ds(base, 128), :], rows0, rs[0])
    rd[1] = pltpu.async_copy(acc_sh.at[pl.ds(base + 128, 128), :], rows1,
                             rs[1])
    for t in range(nrd):
        rb = (rows0, rows1)[t % 2]
        rd[t].wait()
        wr[t] = pltpu.async_copy(
            rb, out_hbm.at[cid, pl.ds(base + t * 128, 128), :], rs[2 + t % 2])
        if t + 2 < nrd:
            wr[t].wait()
            rd[t + 2] = pltpu.async_copy(
                acc_sh.at[pl.ds(base + (t + 2) * 128, 128), :], rb, rs[t % 2])
    wr[nrd - 2].wait()
    wr[nrd - 1].wait()


def _split_store(hs_ref, hsd):
    hs_ref[0, pl.ds(0, N), :] = hsd[:, :HD]
    hs_ref[1, pl.ds(0, N), :] = hsd[:, HD:]
    z = jnp.zeros((R - N, HD), jnp.float32)
    hs_ref[0, pl.ds(N, R - N), :] = z
    hs_ref[1, pl.ds(N, R - N), :] = z


def _tc_first_body(x_ref, w_ref, degt_ref, hs_ref, dinv_ref):
    deg = degt_ref[...]
    s = deg[:N, 0:1] + deg[:N, 1:2] + 1.0
    dinv = lax.rsqrt(s)
    dinv_ref[...] = dinv
    h = jnp.dot(x_ref[...], w_ref[...], preferred_element_type=jnp.float32)
    _split_store(hs_ref, h * dinv)


_tc_first = pl.pallas_call(
    _tc_first_body,
    out_shape=[
        jax.ShapeDtypeStruct((NC, R, HD), jnp.float32),
        jax.ShapeDtypeStruct((N, 1), jnp.float32),
    ],
)


def _gcn_out(a_ref, hs_ref, dinv, b_ref):
    o = jnp.concatenate(
        [a_ref[0, :N, :] + hs_ref[0, :N, :],
         a_ref[1, :N, :] + hs_ref[1, :N, :]], axis=1)
    return o * dinv + b_ref[...]


def _tc_mid_body(a_ref, hs_ref, dinv_ref, b_ref, g_ref, be_ref, w_ref,
                 hs2_ref):
    dinv = dinv_ref[...]
    o = _gcn_out(a_ref, hs_ref, dinv, b_ref)
    mu = jnp.mean(o, axis=0, keepdims=True)
    xc = o - mu
    var = jnp.mean(xc * xc, axis=0, keepdims=True)
    xn = g_ref[...] * xc * lax.rsqrt(var + 1e-5) + be_ref[...]
    h2 = jnp.dot(xn, w_ref[...], preferred_element_type=jnp.float32)
    _split_store(hs2_ref, h2 * dinv)


_tc_mid = pl.pallas_call(
    _tc_mid_body,
    out_shape=[jax.ShapeDtypeStruct((NC, R, HD), jnp.float32)],
)


def _tc_last_body(a_ref, hs_ref, dinv_ref, b_ref, out_ref, h_ref):
    h = _gcn_out(a_ref, hs_ref, dinv_ref[...], b_ref)
    m = jnp.max(h, axis=1, keepdims=True)
    e = jnp.exp(h - m)
    lse = jnp.log(jnp.sum(e, axis=1, keepdims=True)) + m
    out_ref[...] = h - lse
    h_ref[...] = h


_tc_last = pl.pallas_call(
    _tc_last_body,
    out_shape=[
        jax.ShapeDtypeStruct((N, D), jnp.float32),
        jax.ShapeDtypeStruct((N, D), jnp.float32),
    ],
)


def kernel(x, edge_index, W1, b1, W2, b2, W3, b3, g1, be1, g2, be2):
    npad = EPT - E // NS
    pad = (N + jnp.arange(npad, dtype=jnp.int32) % (R - N))[None, :]
    pad = jnp.broadcast_to(pad, (NS, npad))
    src = jnp.concatenate([edge_index[0].reshape(NS, E // NS), pad], axis=1)
    dst = jnp.concatenate([edge_index[1].reshape(NS, E // NS), pad], axis=1)
    srcb = src.reshape(NS, CH, 128)
    dstb = dst.reshape(NS, CH, 128)
    srcb2 = jnp.stack([srcb, srcb + R])
    dstb_deg = dstb.reshape(NW, CHD, 128)

    degp = _sc_deg(dstb_deg)
    hs1, dinv = _tc_first(x, W1, degp.T)
    a = _sc_prop(hs1.reshape(NC * R, HD), srcb2, dstb)
    (hs2,) = _tc_mid(a, hs1, dinv, b1.reshape(1, D), g1.reshape(1, D),
                     be1.reshape(1, D), W2)
    a = _sc_prop(hs2.reshape(NC * R, HD), srcb2, dstb)
    (hs3,) = _tc_mid(a, hs2, dinv, b2.reshape(1, D), g2.reshape(1, D),
                     be2.reshape(1, D), W3)
    a = _sc_prop(hs3.reshape(NC * R, HD), srcb2, dstb)
    out, h = _tc_last(a, hs3, dinv, b3.reshape(1, D))
    return (out, h)


# speedup vs baseline: 1.0286x; 1.0286x over previous
import functools

import jax
import jax.numpy as jnp
from jax import lax
from jax.experimental import pallas as pl
from jax.experimental.pallas import tpu as pltpu
from jax.experimental.pallas import tpu_sc as plsc

N = 10000
E = 320000
D = 128
HD = D // 2

NC = 2
NS = 16
NW = NC * NS
R = 10240
STRIPE = R // NS

EPT = 20480
CH = EPT // 128
CHD = (EPT * NS) // (NW * 128)

_mesh = plsc.VectorSubcoreMesh(core_axis_name="c", subcore_axis_name="s")


@functools.partial(
    pl.kernel,
    out_type=jax.ShapeDtypeStruct((NC, R), jnp.float32),
    mesh=_mesh,
    scratch_types=[
        pltpu.VMEM((CHD, 128), jnp.int32),
        pltpu.VMEM((STRIPE,), jnp.float32),
        pltpu.VMEM((128,), jnp.float32),
        pltpu.VMEM_SHARED((R,), jnp.float32),
    ],
)
def _sc_deg(dstb_hbm, out_hbm, dst_v, stage_v, ones_v, deg_sh):
    cid = lax.axis_index("c")
    sid = lax.axis_index("s")
    wid = cid * NS + sid
    pltpu.sync_copy(dstb_hbm.at[wid], dst_v)

    def _zero(i, _):
        stage_v[pl.ds(i * 16, 16)] = jnp.zeros((16,), jnp.float32)
        return 0

    lax.fori_loop(0, STRIPE // 16, _zero, 0)
    for k in range(8):
        ones_v[pl.ds(k * 16, 16)] = jnp.ones((16,), jnp.float32)
    pltpu.sync_copy(stage_v, deg_sh.at[pl.ds(sid * STRIPE, STRIPE)])
    plsc.subcore_barrier()

    def _body(j, _):
        pltpu.sync_copy(ones_v, deg_sh.at[dst_v.at[j]], add=True)
        return 0

    lax.fori_loop(0, CHD, _body, 0)
    plsc.subcore_barrier()
    pltpu.sync_copy(deg_sh.at[pl.ds(sid * STRIPE, STRIPE)], stage_v)
    pltpu.sync_copy(stage_v, out_hbm.at[cid, pl.ds(sid * STRIPE, STRIPE)])


@functools.partial(
    pl.kernel,
    out_type=jax.ShapeDtypeStruct((NC, R, HD), jnp.float32),
    mesh=_mesh,
    scratch_types=[
        pltpu.VMEM((CH, 128), jnp.int32),
        pltpu.VMEM((CH, 128), jnp.int32),
        pltpu.VMEM((128, HD), jnp.float32),
        pltpu.VMEM((128, HD), jnp.float32),
        pltpu.VMEM((128, HD), jnp.float32),
        pltpu.VMEM((128, HD), jnp.float32),
        pltpu.VMEM((128, HD), jnp.float32),
        pltpu.VMEM_SHARED((R, HD), jnp.float32),
        [pltpu.SemaphoreType.DMA] * 5,
        [pltpu.SemaphoreType.DMA] * 5,
        [pltpu.SemaphoreType.DMA] * 4,
    ],
    compiler_params=pltpu.CompilerParams(use_tc_tiling_on_sc=False),
)
def _sc_prop(hs_hbm, srcb_hbm, dstb_hbm, out_hbm, src_v, dst_v, rows0, rows1,
             rows2, rows3, rows4, acc_sh, sg, ss, rs):
    cid = lax.axis_index("c")
    sid = lax.axis_index("s")
    pltpu.sync_copy(srcb_hbm.at[cid, sid], src_v)
    pltpu.sync_copy(dstb_hbm.at[sid], dst_v)

    def _zero(i, _):
        for k in range(HD // 16):
            rows0[i, pl.ds(k * 16, 16)] = jnp.zeros((16,), jnp.float32)
        return 0

    lax.fori_loop(0, 128, _zero, 0)
    base = sid * STRIPE
    for t in range(STRIPE // 128):
        pltpu.sync_copy(rows0, acc_sh.at[pl.ds(base + t * 128, 128), :])
    plsc.subcore_barrier()

    rows = (rows0, rows1, rows2, rows3, rows4)
    NB = 5
    LAG = 2

    def _gather(t, b):
        return pltpu.async_copy(hs_hbm.at[src_v.at[t]], rows[b], sg[b])

    def _wait_gather(t, b):
        pltpu.make_async_copy(hs_hbm.at[src_v.at[t]], rows[b], sg[b]).wait()

    def _scatter(t, b):
        return pltpu.async_copy(rows[b], acc_sh.at[dst_v.at[t]], ss[b],
                                add=True)

    def _wait_scatter(t, b):
        pltpu.make_async_copy(rows[b], acc_sh.at[dst_v.at[t]], ss[b]).wait()

    for b in range(NB):
        _gather(b, b)
        if b >= LAG:
            _wait_gather(b - LAG, b - LAG)
            _scatter(b - LAG, b - LAG)

    def _body(i, _):
        g = NB * i
        for b in range(NB):
            t = g + b
            b2 = (b + NB - LAG) % NB
            _wait_scatter(t - NB, b)
            _gather(t, b)
            _wait_gather(t - LAG, b2)
            _scatter(t - LAG, b2)
        return 0

    lax.fori_loop(1, CH // NB, _body, 0)
    for t in range(CH - LAG, CH):
        _wait_gather(t, t % NB)
        _scatter(t, t % NB)
    for b in range(NB):
        _wait_scatter(CH - NB + b, b)
    plsc.subcore_barrier()

    nrd = STRIPE // 128
    rd = [None] * nrd
    wr = [None] * nrd
    rd[0] = pltpu.async_copy(acc_sh.at[pl.ds(base, 128), :], rows0, rs[0])
    rd[1] = pltpu.async_copy(acc_sh.at[pl.ds(base + 128, 128), :], rows1,
                             rs[1])
    for t in range(nrd):
        rb = (rows0, rows1)[t % 2]
        rd[t].wait()
        wr[t] = pltpu.async_copy(
            rb, out_hbm.at[cid, pl.ds(base + t * 128, 128), :], rs[2 + t % 2])
        if t + 2 < nrd:
            wr[t].wait()
            rd[t + 2] = pltpu.async_copy(
                acc_sh.at[pl.ds(base + (t + 2) * 128, 128), :], rb, rs[t % 2])
    wr[nrd - 2].wait()
    wr[nrd - 1].wait()


def _split_store(hs_ref, hsd):
    hs_ref[0, pl.ds(0, N), :] = hsd[:, :HD]
    hs_ref[1, pl.ds(0, N), :] = hsd[:, HD:]
    z = jnp.zeros((R - N, HD), jnp.float32)
    hs_ref[0, pl.ds(N, R - N), :] = z
    hs_ref[1, pl.ds(N, R - N), :] = z


def _tc_first_body(x_ref, w_ref, degt_ref, hs_ref, dinv_ref):
    deg = degt_ref[...]
    s = deg[:N, 0:1] + deg[:N, 1:2] + 1.0
    dinv = lax.rsqrt(s)
    dinv_ref[...] = dinv
    h = jnp.dot(x_ref[...], w_ref[...], preferred_element_type=jnp.float32)
    _split_store(hs_ref, h * dinv)


_tc_first = pl.pallas_call(
    _tc_first_body,
    out_shape=[
        jax.ShapeDtypeStruct((NC, R, HD), jnp.float32),
        jax.ShapeDtypeStruct((N, 1), jnp.float32),
    ],
)


def _gcn_out(a_ref, hs_ref, dinv, b_ref):
    o = jnp.concatenate(
        [a_ref[0, :N, :] + hs_ref[0, :N, :],
         a_ref[1, :N, :] + hs_ref[1, :N, :]], axis=1)
    return o * dinv + b_ref[...]


def _tc_mid_body(a_ref, hs_ref, dinv_ref, b_ref, g_ref, be_ref, w_ref,
                 hs2_ref):
    dinv = dinv_ref[...]
    o = _gcn_out(a_ref, hs_ref, dinv, b_ref)
    mu = jnp.mean(o, axis=0, keepdims=True)
    xc = o - mu
    var = jnp.mean(xc * xc, axis=0, keepdims=True)
    xn = g_ref[...] * xc * lax.rsqrt(var + 1e-5) + be_ref[...]
    h2 = jnp.dot(xn, w_ref[...], preferred_element_type=jnp.float32)
    _split_store(hs2_ref, h2 * dinv)


_tc_mid = pl.pallas_call(
    _tc_mid_body,
    out_shape=[jax.ShapeDtypeStruct((NC, R, HD), jnp.float32)],
)


def _tc_last_body(a_ref, hs_ref, dinv_ref, b_ref, out_ref, h_ref):
    h = _gcn_out(a_ref, hs_ref, dinv_ref[...], b_ref)
    m = jnp.max(h, axis=1, keepdims=True)
    e = jnp.exp(h - m)
    lse = jnp.log(jnp.sum(e, axis=1, keepdims=True)) + m
    out_ref[...] = h - lse
    h_ref[...] = h


_tc_last = pl.pallas_call(
    _tc_last_body,
    out_shape=[
        jax.ShapeDtypeStruct((N, D), jnp.float32),
        jax.ShapeDtypeStruct((N, D), jnp.float32),
    ],
)


def kernel(x, edge_index, W1, b1, W2, b2, W3, b3, g1, be1, g2, be2):
    npad = EPT - E // NS
    pad = (N + jnp.arange(npad, dtype=jnp.int32) % (R - N))[None, :]
    pad = jnp.broadcast_to(pad, (NS, npad))
    src = jnp.concatenate([edge_index[0].reshape(NS, E // NS), pad], axis=1)
    dst = jnp.concatenate([edge_index[1].reshape(NS, E // NS), pad], axis=1)
    srcb = src.reshape(NS, CH, 128)
    dstb = dst.reshape(NS, CH, 128)
    srcb2 = jnp.stack([srcb, srcb + R])
    dstb_deg = dstb.reshape(NW, CHD, 128)

    degp = _sc_deg(dstb_deg)
    hs1, dinv = _tc_first(x, W1, degp.T)
    a = _sc_prop(hs1.reshape(NC * R, HD), srcb2, dstb)
    (hs2,) = _tc_mid(a, hs1, dinv, b1.reshape(1, D), g1.reshape(1, D),
                     be1.reshape(1, D), W2)
    a = _sc_prop(hs2.reshape(NC * R, HD), srcb2, dstb)
    (hs3,) = _tc_mid(a, hs2, dinv, b2.reshape(1, D), g2.reshape(1, D),
                     be2.reshape(1, D), W3)
    a = _sc_prop(hs3.reshape(NC * R, HD), srcb2, dstb)
    out, h = _tc_last(a, hs3, dinv, b3.reshape(1, D))
    return (out, h)
